# Initial kernel scaffold; baseline (speedup 1.0000x reference)
import jax
import jax.numpy as jnp
from jax.experimental import pallas as pl


def kernel(sigma, sigmas):
    raise NotImplementedError("write your pallas kernel here")


# speedup vs baseline: 53.1765x; 53.1765x over previous
import functools

import jax
import jax.numpy as jnp
from jax import lax
from jax.experimental import pallas as pl
from jax.experimental.pallas import tpu as pltpu
from jax.experimental.pallas import tpu_sc as plsc

N_QUERIES = 65536
N_LEVELS = 1000
NC = 2
NS = 16
L = 16
NW = NC * NS
QPW = N_QUERIES // NW
NV = QPW // L
TAB_PAD = 1008

_P0 = -2.4967665255106644
_P1 = 4.0283552158829
_P2 = -2.0810447771259137
_P3 = 0.6288099281987508
_P4 = -0.07914958442881646


def _cf(v):
    return jnp.full((L,), v, jnp.float32)


def _ci(v):
    return jnp.full((L,), v, jnp.int32)


_mesh = plsc.VectorSubcoreMesh(core_axis_name="c", subcore_axis_name="s")


@functools.partial(
    pl.kernel,
    mesh=_mesh,
    out_type=jax.ShapeDtypeStruct((N_QUERIES,), jnp.float32),
    compiler_params=pltpu.CompilerParams(needs_layout_passes=False),
    scratch_types=[
        pltpu.VMEM((QPW,), jnp.float32),
        pltpu.VMEM((TAB_PAD,), jnp.float32),
        pltpu.VMEM((TAB_PAD,), jnp.float32),
        pltpu.VMEM((2, L), jnp.float32),
        pltpu.VMEM((QPW,), jnp.float32),
    ],
)
def _sigma_to_t_sc(sigma_hbm, tab_hbm, invd_hbm, cst_hbm, out_hbm,
                   sig_v, tab_v, invd_v, cst_v, out_v):
    wid = lax.axis_index("s") * NC + lax.axis_index("c")
    base = wid * QPW
    pltpu.sync_copy(sigma_hbm.at[pl.ds(base, QPW)], sig_v)
    pltpu.sync_copy(tab_hbm, tab_v)
    pltpu.sync_copy(invd_hbm, invd_v)
    pltpu.sync_copy(cst_hbm, cst_v)
    a_v = cst_v[0, :]
    b_v = cst_v[1, :]

    def body(i, carry):
        s = sig_v[pl.ds(i * L, L)]
        bits = lax.bitcast_convert_type(s, jnp.int32)
        e_f = lax.convert_element_type(
            lax.shift_right_arithmetic(bits, _ci(23)) - _ci(127), jnp.float32)
        m = lax.bitcast_convert_type(
            (bits & _ci(0x7FFFFF)) | _ci(0x3F800000), jnp.float32)
        p = _cf(_P4)
        p = p * m + _cf(_P3)
        p = p * m + _cf(_P2)
        p = p * m + _cf(_P1)
        p = p * m + _cf(_P0)
        y = e_f + p
        g = y * a_v + b_v
        g = jnp.minimum(jnp.maximum(g, _cf(0.0)), _cf(998.0))
        idx0 = lax.convert_element_type(g, jnp.int32)
        g0 = plsc.load_gather(tab_v, [idx0])
        g1 = plsc.load_gather(tab_v, [idx0 + _ci(1)])
        idx = jnp.where(s < g0, idx0 - _ci(1),
                        jnp.where(s >= g1, idx0 + _ci(1), idx0))
        idx = jnp.minimum(jnp.maximum(idx, _ci(0)), _ci(998))
        low = plsc.load_gather(tab_v, [idx])
        vd = plsc.load_gather(invd_v, [idx])
        u = s / low - _cf(1.0)
        u = jnp.minimum(jnp.maximum(u, _cf(-0.25)), _cf(0.25))
        r = u * (_cf(1.0) + u * (_cf(-0.5) + u * _cf(1.0 / 3.0)))
        w = jnp.minimum(jnp.maximum(r * vd, _cf(0.0)), _cf(1.0))
        out_v[pl.ds(i * L, L)] = lax.convert_element_type(idx, jnp.float32) + w
        return carry

    lax.fori_loop(0, NV, body, 0)
    pltpu.sync_copy(out_v, out_hbm.at[pl.ds(base, QPW)])


def kernel(sigma, sigmas):
    sigmas = sigmas.astype(jnp.float32)
    log_s = jnp.log(sigmas)
    invd = 1.0 / (log_s[1:] - log_s[:-1])
    invd = jnp.concatenate(
        [invd, jnp.ones((TAB_PAD - (N_LEVELS - 1),), jnp.float32)])
    tab = jnp.concatenate(
        [sigmas, jnp.full((TAB_PAD - N_LEVELS,), jnp.inf, jnp.float32)])
    l2lo = jnp.log2(sigmas[0])
    l2hi = jnp.log2(sigmas[-1])
    a = (N_LEVELS - 1) / (l2hi - l2lo)
    b = -l2lo * a
    cst = jnp.stack([jnp.full((L,), a, jnp.float32),
                     jnp.full((L,), b, jnp.float32)])
    t = _sigma_to_t_sc(sigma.reshape(N_QUERIES).astype(jnp.float32),
                       tab, invd, cst)
    return t.reshape(sigma.shape)
